# baseline (speedup 1.0000x reference)
import jax
import jax.numpy as jnp
from jax.experimental import pallas as pl


def _copy_kernel(edges_ref, out_ref):
    out_ref[...] = edges_ref[...]


def kernel(node_features, edges, layer_lengths, W1, b1, W2, b2):
    n = edges.shape[0] * edges.shape[1]
    flat = edges.reshape(n // 128, 128)
    out = pl.pallas_call(
        _copy_kernel,
        out_shape=jax.ShapeDtypeStruct(flat.shape, flat.dtype),
    )(flat)
    return out.reshape(edges.shape)
